# baseline (speedup 1.0000x reference)
import functools

import jax
import jax.numpy as jnp
from jax import lax
from jax.experimental import pallas as pl
from jax.experimental.pallas import tpu as pltpu
from jax.experimental.pallas import tpu_sc as plsc

_LANES = 128
_K = 8
_CHUNK = _K * _LANES


@functools.lru_cache(maxsize=None)
def _build(n_sub: int, d: int):
    info = plsc.get_sparse_core_info()
    nc, ns = info.num_cores, info.num_subcores
    nw = nc * ns
    sub_per_w = n_sub // nw
    n_chunks = sub_per_w // _K
    assert sub_per_w * nw == n_sub and n_chunks * _K == sub_per_w

    mesh = plsc.VectorSubcoreMesh(core_axis_name="c", subcore_axis_name="s")

    @functools.partial(
        pl.kernel,
        mesh=mesh,
        out_type=jax.ShapeDtypeStruct((n_sub * _LANES, d), jnp.float32),
        scratch_types=[
            pltpu.VMEM((_K, _LANES), jnp.int32),
            pltpu.VMEM((_CHUNK, d), jnp.float32),
            pltpu.SemaphoreType.DMA,
        ],
        compiler_params=pltpu.CompilerParams(use_tc_tiling_on_sc=False),
    )
    def gather_kernel(table_hbm, idx_hbm, out_hbm, idx_v, rows_v, sem):
        wid = lax.axis_index("s") * nc + lax.axis_index("c")
        sub_base = wid * sub_per_w

        def chunk_body(g, carry):
            sub0 = sub_base + g * _K
            pltpu.sync_copy(idx_hbm.at[pl.ds(sub0, _K)], idx_v)
            copies = [
                pltpu.async_copy(
                    table_hbm.at[idx_v.at[j]],
                    rows_v.at[pl.ds(j * _LANES, _LANES)],
                    sem,
                )
                for j in range(_K)
            ]
            for c in copies:
                c.wait()
            pltpu.sync_copy(rows_v, out_hbm.at[pl.ds(sub0 * _LANES, _CHUNK)])
            return carry

        lax.fori_loop(0, n_chunks, chunk_body, 0)

    return gather_kernel


def kernel(x, token_table):
    b, t = x.shape
    d = token_table.shape[1]
    n = b * t
    idx2d = x.reshape(n // _LANES, _LANES)
    out = _build(n // _LANES, d)(token_table, idx2d)
    return out.reshape(b, t, d)


# speedup vs baseline: 1.0196x; 1.0196x over previous
import functools

import jax
import jax.numpy as jnp
from jax import lax
from jax.experimental import pallas as pl
from jax.experimental.pallas import tpu as pltpu
from jax.experimental.pallas import tpu_sc as plsc

_LANES = 128
_K = 4
_CHUNK = _K * _LANES


@functools.lru_cache(maxsize=None)
def _build(n_sub: int, d: int):
    info = plsc.get_sparse_core_info()
    nc, ns = info.num_cores, info.num_subcores
    nw = nc * ns
    sub_per_w = n_sub // nw
    n_chunks = sub_per_w // _K
    assert sub_per_w * nw == n_sub and n_chunks * _K == sub_per_w
    assert n_chunks % 2 == 0 and n_chunks >= 4

    mesh = plsc.VectorSubcoreMesh(core_axis_name="c", subcore_axis_name="s")

    @functools.partial(
        pl.kernel,
        mesh=mesh,
        out_type=jax.ShapeDtypeStruct((n_sub * _LANES, d), jnp.float32),
        scratch_types=[
            pltpu.VMEM((sub_per_w, _LANES), jnp.int32),
            pltpu.VMEM((_CHUNK, d), jnp.float32),
            pltpu.VMEM((_CHUNK, d), jnp.float32),
            pltpu.SemaphoreType.DMA,
            pltpu.SemaphoreType.DMA,
        ],
        compiler_params=pltpu.CompilerParams(use_tc_tiling_on_sc=False),
    )
    def gather_kernel(table_hbm, idx_hbm, out_hbm, idx_v, rows0, rows1,
                      sem0, sem1):
        wid = lax.axis_index("s") * nc + lax.axis_index("c")
        sub_base = wid * sub_per_w
        rows = (rows0, rows1)
        sems = (sem0, sem1)

        pltpu.sync_copy(idx_hbm.at[pl.ds(sub_base, sub_per_w)], idx_v)

        def fire(g, b):
            for j in range(_K):
                pltpu.make_async_copy(
                    table_hbm.at[idx_v.at[g * _K + j]],
                    rows[b].at[pl.ds(j * _LANES, _LANES)],
                    sems[b],
                ).start()

        def drain(g, b):
            for j in range(_K):
                pltpu.make_async_copy(
                    table_hbm.at[idx_v.at[g * _K + j]],
                    rows[b].at[pl.ds(j * _LANES, _LANES)],
                    sems[b],
                ).wait()

        fire(0, 0)
        fire(1, 1)

        def body(i, carry):
            for b in range(2):
                g = 2 * i + b
                drain(g, b)
                pltpu.sync_copy(
                    rows[b],
                    out_hbm.at[pl.ds((sub_base + g * _K) * _LANES, _CHUNK)],
                )

                @pl.when(g + 2 < n_chunks)
                def _():
                    fire(g + 2, b)

            return carry

        lax.fori_loop(0, n_chunks // 2, body, 0)

    return gather_kernel


def kernel(x, token_table):
    b, t = x.shape
    d = token_table.shape[1]
    n = b * t
    idx2d = x.reshape(n // _LANES, _LANES)
    out = _build(n // _LANES, d)(token_table, idx2d)
    return out.reshape(b, t, d)


# speedup vs baseline: 1.3521x; 1.3261x over previous
import functools

import jax
import jax.numpy as jnp
from jax import lax
from jax.experimental import pallas as pl
from jax.experimental.pallas import tpu as pltpu
from jax.experimental.pallas import tpu_sc as plsc

_LANES = 128
_K = 4
_CHUNK = _K * _LANES


@functools.lru_cache(maxsize=None)
def _build(n_sub: int, d: int):
    info = plsc.get_sparse_core_info()
    nc, ns = info.num_cores, info.num_subcores
    nw = nc * ns
    sub_per_w = n_sub // nw
    n_chunks = sub_per_w // _K
    assert sub_per_w * nw == n_sub and n_chunks * _K == sub_per_w
    assert n_chunks % 2 == 0 and n_chunks >= 4

    mesh = plsc.VectorSubcoreMesh(core_axis_name="c", subcore_axis_name="s")

    @functools.partial(
        pl.kernel,
        mesh=mesh,
        out_type=jax.ShapeDtypeStruct((n_sub * _LANES, 2 * d), jnp.float32),
        scratch_types=[
            pltpu.VMEM((sub_per_w, _LANES), jnp.int32),
            pltpu.VMEM((_CHUNK, d), jnp.float32),
            pltpu.VMEM((_CHUNK, d), jnp.float32),
            pltpu.SemaphoreType.DMA,
            pltpu.SemaphoreType.DMA,
        ],
        compiler_params=pltpu.CompilerParams(use_tc_tiling_on_sc=False),
    )
    def gather_kernel(table_hbm, idx_hbm, out_hbm, idx_v, rows0, rows1,
                      sem0, sem1):
        wid = lax.axis_index("s") * nc + lax.axis_index("c")
        sub_base = wid * sub_per_w
        rows = (rows0, rows1)
        sems = (sem0, sem1)

        pltpu.sync_copy(idx_hbm.at[pl.ds(sub_base, sub_per_w)], idx_v)

        def fire(g, b):
            for j in range(_K):
                pltpu.make_async_copy(
                    table_hbm.at[idx_v.at[g * _K + j]],
                    rows[b].at[pl.ds(j * _LANES, _LANES)],
                    sems[b],
                ).start()

        def drain(g, b):
            for j in range(_K):
                pltpu.make_async_copy(
                    table_hbm.at[idx_v.at[g * _K + j]],
                    rows[b].at[pl.ds(j * _LANES, _LANES)],
                    sems[b],
                ).wait()

        fire(0, 0)
        fire(1, 1)

        def body(i, carry):
            for b in range(2):
                g = 2 * i + b
                drain(g, b)
                pltpu.sync_copy(
                    rows[b],
                    out_hbm.at[
                        pl.ds((sub_base + g * _K) * _LANES, _CHUNK),
                        pl.ds(0, d),
                    ],
                )

                @pl.when(g + 2 < n_chunks)
                def _():
                    fire(g + 2, b)

            return carry

        lax.fori_loop(0, n_chunks // 2, body, 0)

    return gather_kernel


def kernel(x, token_table):
    b, t = x.shape
    d = token_table.shape[1]
    n = b * t
    idx2d = x.reshape(n // _LANES, _LANES)
    out2 = _build(n // _LANES, d)(token_table, idx2d)
    return out2.reshape(b, t, 2 * d)[:, :, :d]
